# Initial kernel scaffold; baseline (speedup 1.0000x reference)
import jax
import jax.numpy as jnp
from jax.experimental import pallas as pl


def kernel(seq_len, table):
    raise NotImplementedError("write your pallas kernel here")


# speedup vs baseline: 8.1419x; 8.1419x over previous
import functools

import jax
import jax.numpy as jnp
from jax import lax
from jax.experimental import pallas as pl
from jax.experimental.pallas import tpu as pltpu
from jax.experimental.pallas import tpu_sc as plsc

D_MODEL = 64
MAX_REL = 32
VOCAB = 2 * MAX_REL + 1
SEQ = 2048
NUM_CORES = 2
NUM_SUBCORES = 16
NW = NUM_CORES * NUM_SUBCORES
ROWS_PER_W = SEQ // NW
NST = SEQ // 128
GV_H = 448
PREF = 255
Q_MAX = GV_H - 128


@functools.partial(
    pl.kernel,
    mesh=plsc.VectorSubcoreMesh(core_axis_name="c", subcore_axis_name="s"),
    out_type=jax.ShapeDtypeStruct((SEQ, SEQ, D_MODEL), jnp.float32),
    scratch_types=[
        pltpu.VMEM((VOCAB, D_MODEL), jnp.float32),
        pltpu.VMEM((GV_H, D_MODEL), jnp.float32),
        pltpu.SemaphoreType.DMA,
        pltpu.SemaphoreType.DMA,
    ],
)
def _rel_pos_sc(table_hbm, out_hbm, tbl_v, gv_v, sem_a, sem_b):
    cid = lax.axis_index("c")
    sid = lax.axis_index("s")
    wid = sid * NUM_CORES + cid
    row0 = wid * ROWS_PER_W

    pltpu.sync_copy(table_hbm, tbl_v)

    def build_body(m, carry):
        src_row = jnp.clip(m - PREF, 0, VOCAB - 1)
        for c in range(D_MODEL // 16):
            gv_v[m, pl.ds(c * 16, 16)] = tbl_v[src_row, pl.ds(c * 16, 16)]
        return carry

    lax.fori_loop(0, GV_H, build_body, 0)

    def row_copies(i, sem):
        descs = []
        for k in range(NST):
            m0 = jnp.clip(128 * k - i + 287, 0, Q_MAX)
            descs.append(
                pltpu.make_async_copy(
                    gv_v.at[pl.ds(m0, 128)],
                    out_hbm.at[i, pl.ds(128 * k, 128)],
                    sem,
                )
            )
        return descs

    def issue_row(i, sem):
        for d in row_copies(i, sem):
            d.start()

    def wait_row(i, sem):
        for d in row_copies(i, sem):
            d.wait()

    def pair_body(t, carry):
        i0 = row0 + 2 * t
        issue_row(i0, sem_a)

        @pl.when(t > 0)
        def _wait_prev_odd():
            wait_row(i0 - 1, sem_b)

        issue_row(i0 + 1, sem_b)
        wait_row(i0, sem_a)
        return carry

    lax.fori_loop(0, ROWS_PER_W // 2, pair_body, 0)
    wait_row(row0 + ROWS_PER_W - 1, sem_b)


def kernel(seq_len, table):
    del seq_len
    return _rel_pos_sc(table)


# speedup vs baseline: 8.1669x; 1.0031x over previous
import functools

import jax
import jax.numpy as jnp
from jax import lax
from jax.experimental import pallas as pl
from jax.experimental.pallas import tpu as pltpu
from jax.experimental.pallas import tpu_sc as plsc

D_MODEL = 64
MAX_REL = 32
VOCAB = 2 * MAX_REL + 1
SEQ = 2048
NUM_CORES = 2
NUM_SUBCORES = 16
NW = NUM_CORES * NUM_SUBCORES
ROWS_PER_W = SEQ // NW
BLK = 256
NBLK = SEQ // BLK
GV_H = BLK + VOCAB + BLK
Q_MAX = GV_H - BLK


@functools.partial(
    pl.kernel,
    mesh=plsc.VectorSubcoreMesh(core_axis_name="c", subcore_axis_name="s"),
    out_type=jax.ShapeDtypeStruct((SEQ, SEQ, D_MODEL), jnp.float32),
    scratch_types=[
        pltpu.VMEM((GV_H, D_MODEL), jnp.float32),
        pltpu.SemaphoreType.DMA,
        pltpu.SemaphoreType.DMA,
    ],
)
def _rel_pos_sc(table_hbm, out_hbm, gv_v, sem_a, sem_b):
    cid = lax.axis_index("c")
    sid = lax.axis_index("s")
    wid = sid * NUM_CORES + cid
    row0 = wid * ROWS_PER_W

    pltpu.sync_copy(table_hbm, gv_v.at[pl.ds(BLK, VOCAB)])

    v_first = [gv_v[BLK, pl.ds(c * 16, 16)] for c in range(D_MODEL // 16)]
    v_last = [
        gv_v[BLK + VOCAB - 1, pl.ds(c * 16, 16)] for c in range(D_MODEL // 16)
    ]

    def fill_body(m, carry):
        for c in range(D_MODEL // 16):
            gv_v[m, pl.ds(c * 16, 16)] = v_first[c]
            gv_v[BLK + VOCAB + m, pl.ds(c * 16, 16)] = v_last[c]
        return carry

    lax.fori_loop(0, BLK, fill_body, 0)

    def row_copies(i, sem):
        descs = []
        for k in range(NBLK):
            m0 = jnp.clip(BLK * k - i + (2047 - (2015 - BLK)), 0, Q_MAX)
            descs.append(
                pltpu.make_async_copy(
                    gv_v.at[pl.ds(m0, BLK)],
                    out_hbm.at[i, pl.ds(BLK * k, BLK)],
                    sem,
                )
            )
        return descs

    def issue_row(i, sem):
        for d in row_copies(i, sem):
            d.start()

    def wait_row(i, sem):
        for d in row_copies(i, sem):
            d.wait()

    def pair_body(t, carry):
        i0 = row0 + 2 * t
        issue_row(i0, sem_a)

        @pl.when(t > 0)
        def _wait_prev_odd():
            wait_row(i0 - 1, sem_b)

        issue_row(i0 + 1, sem_b)
        wait_row(i0, sem_a)
        return carry

    lax.fori_loop(0, ROWS_PER_W // 2, pair_body, 0)
    wait_row(row0 + ROWS_PER_W - 1, sem_b)


def kernel(seq_len, table):
    del seq_len
    return _rel_pos_sc(table)


# speedup vs baseline: 9.1921x; 1.1255x over previous
import functools

import jax
import jax.numpy as jnp
from jax import lax
from jax.experimental import pallas as pl
from jax.experimental.pallas import tpu as pltpu
from jax.experimental.pallas import tpu_sc as plsc

D_MODEL = 64
MAX_REL = 32
VOCAB = 2 * MAX_REL + 1
SEQ = 2048
NUM_CORES = 2
NUM_SUBCORES = 16
NW = NUM_CORES * NUM_SUBCORES
ROWS_PER_W = SEQ // NW
NST = SEQ // 128
GT_W = 512
Q_MAX = 320


def _build_gt8_tc(table_ref, out_ref):
    tbl = table_ref[:]
    rows = lax.broadcasted_iota(jnp.int32, (VOCAB, GT_W), 0)
    m = lax.broadcasted_iota(jnp.int32, (VOCAB, GT_W), 1)
    for s in range(8):
        src = jnp.clip(s + m - 255, 0, VOCAB - 1)
        onehot = (rows == src).astype(jnp.float32)
        out_ref[s] = lax.dot_general(
            tbl, onehot, (((0,), (0,)), ((), ())),
            preferred_element_type=jnp.float32,
        )


_gt8 = pl.pallas_call(
    _build_gt8_tc,
    out_shape=jax.ShapeDtypeStruct((8, D_MODEL, GT_W), jnp.float32),
)


@functools.partial(
    pl.kernel,
    mesh=plsc.VectorSubcoreMesh(core_axis_name="c", subcore_axis_name="s"),
    out_type=jax.ShapeDtypeStruct((SEQ, NST, D_MODEL, 128), jnp.float32),
    scratch_types=[
        pltpu.VMEM((D_MODEL, GT_W), jnp.float32),
        pltpu.SemaphoreType.DMA,
        pltpu.SemaphoreType.DMA,
    ],
    compiler_params=pltpu.CompilerParams(use_tc_tiling_on_sc=False),
)
def _rel_pos_sc(gt8_hbm, out_hbm, gt_v, sem_a, sem_b):
    cid = lax.axis_index("c")
    sid = lax.axis_index("s")
    wid = sid * NUM_CORES + cid
    s0 = (287 - wid) % 8

    pltpu.sync_copy(gt8_hbm.at[s0], gt_v)

    def row_copies(i, sem):
        descs = []
        for k in range(NST):
            m0 = jnp.clip(128 * k - i + 287 - s0, 0, Q_MAX)
            descs.append(
                pltpu.make_async_copy(
                    gt_v.at[:, pl.ds(pl.multiple_of(m0, 8), 128)],
                    out_hbm.at[i, k],
                    sem,
                )
            )
        return descs

    def issue_row(i, sem):
        for d in row_copies(i, sem):
            d.start()

    def wait_row(i, sem):
        for d in row_copies(i, sem):
            d.wait()

    def pair_body(t, carry):
        i0 = wid + NW * (2 * t)
        i1 = wid + NW * (2 * t + 1)
        issue_row(i0, sem_a)

        @pl.when(t > 0)
        def _wait_prev_odd():
            wait_row(i1 - 2 * NW, sem_b)

        issue_row(i1, sem_b)
        wait_row(i0, sem_a)
        return carry

    lax.fori_loop(0, ROWS_PER_W // 2, pair_body, 0)
    wait_row(wid + NW * (ROWS_PER_W - 1), sem_b)


def kernel(seq_len, table):
    del seq_len
    timg = _rel_pos_sc(_gt8(table))
    return timg.transpose(0, 1, 3, 2).reshape(SEQ, SEQ, D_MODEL)
